# Initial kernel scaffold; baseline (speedup 1.0000x reference)
import jax
import jax.numpy as jnp
from jax.experimental import pallas as pl


def kernel(x, edge_index, W1, b1, W2, b2, Wfc, bfc):
    raise NotImplementedError("write your pallas kernel here")


# speedup vs baseline: 34.8731x; 34.8731x over previous
import functools

import jax
import jax.numpy as jnp
from jax import lax
from jax.experimental import pallas as pl
from jax.experimental.pallas import tpu as pltpu
from jax.experimental.pallas import tpu_sc as plsc

N_SH = 10240
W = 128
NC, NS = 2, 16
NW = NC * NS
NWIN = 80
EP = NW * NWIN * W
RPT = N_SH // NS
BN = 1024
GRID = N_SH // BN

_MESH = plsc.VectorSubcoreMesh(core_axis_name="c", subcore_axis_name="s",
                               num_cores=NC, num_subcores=NS)
_SC_PARAMS = pltpu.CompilerParams(use_tc_tiling_on_sc=False)
_f32 = jnp.float32


def _hop_body(g_hbm, z_hbm, ridx_hbm, cidx_hbm, out_hbm,
              ridx_v, cidx_v, rb0, rb1, agg_sh, sem0, sem1):
    c = lax.axis_index("c")
    s = lax.axis_index("s")
    wid = c * NS + s
    r0 = s * RPT

    @pl.when(c == 0)
    def _():
        pltpu.sync_copy(g_hbm.at[pl.ds(r0, RPT)], agg_sh.at[pl.ds(r0, RPT)])

    @pl.when(c != 0)
    def _():
        pltpu.sync_copy(z_hbm.at[pl.ds(r0, RPT)], agg_sh.at[pl.ds(r0, RPT)])

    pltpu.sync_copy(ridx_hbm.at[pl.ds(wid * NWIN, NWIN)], ridx_v)
    pltpu.sync_copy(cidx_hbm.at[pl.ds(wid * NWIN, NWIN)], cidx_v)

    pltpu.async_copy(g_hbm.at[ridx_v.at[0]], rb0, sem0)
    pltpu.async_copy(g_hbm.at[ridx_v.at[1]], rb1, sem1)
    plsc.subcore_barrier()

    rbs = (rb0, rb1)
    sems = (sem0, sem1)

    def outer(i, carry):
        for b in range(2):
            w = 2 * i + b
            pltpu.make_async_copy(g_hbm.at[ridx_v.at[w]], rbs[b],
                                  sems[b]).wait()
            pltpu.sync_copy(rbs[b], agg_sh.at[cidx_v.at[w]], add=True)

            @pl.when(w + 2 < NWIN)
            def _():
                pltpu.async_copy(g_hbm.at[ridx_v.at[w + 2]], rbs[b], sems[b])
        return carry

    lax.fori_loop(0, NWIN // 2, outer, 0)
    plsc.subcore_barrier()
    pltpu.sync_copy(agg_sh.at[pl.ds(r0, RPT)],
                    out_hbm.at[pl.ds(c * N_SH + r0, RPT)])


def _make_hop(F):
    return pl.kernel(
        _hop_body,
        out_type=jax.ShapeDtypeStruct((2 * N_SH, F), _f32),
        mesh=_MESH,
        scratch_types=[
            pltpu.VMEM((NWIN, W), jnp.int32),
            pltpu.VMEM((NWIN, W), jnp.int32),
            pltpu.VMEM((W, F), _f32),
            pltpu.VMEM((W, F), _f32),
            pltpu.VMEM_SHARED((N_SH, F), _f32),
            pltpu.SemaphoreType.DMA,
            pltpu.SemaphoreType.DMA,
        ],
        compiler_params=_SC_PARAMS,
    )


def _deg_body(cidx_hbm, out_hbm, cidx_v, ones_v, zb_v, deg_sh):
    c = lax.axis_index("c")
    s = lax.axis_index("s")
    wid = c * NS + s
    r0 = s * RPT

    for j in range(W // 16):
        ones_v[pl.ds(j * 16, 16)] = jnp.ones((16,), _f32)

    def zf(i, carry):
        zb_v[pl.ds(i * 16, 16)] = jnp.zeros((16,), _f32)
        return carry

    lax.fori_loop(0, RPT // 16, zf, 0)
    pltpu.sync_copy(zb_v, deg_sh.at[pl.ds(r0, RPT)])
    pltpu.sync_copy(cidx_hbm.at[pl.ds(wid * NWIN, NWIN)], cidx_v)
    plsc.subcore_barrier()

    def wf(w, carry):
        pltpu.sync_copy(ones_v, deg_sh.at[cidx_v.at[w]], add=True)
        return carry

    lax.fori_loop(0, NWIN, wf, 0)
    plsc.subcore_barrier()
    pltpu.sync_copy(deg_sh.at[pl.ds(r0, RPT)],
                    out_hbm.at[pl.ds(c * N_SH + r0, RPT)])


_deg_kernel = pl.kernel(
    _deg_body,
    out_type=jax.ShapeDtypeStruct((2 * N_SH,), _f32),
    mesh=_MESH,
    scratch_types=[
        pltpu.VMEM((NWIN, W), jnp.int32),
        pltpu.VMEM((W,), _f32),
        pltpu.VMEM((RPT,), _f32),
        pltpu.VMEM_SHARED((N_SH,), _f32),
    ],
    compiler_params=_SC_PARAMS,
)


def _mm1_body(x_ref, w_ref, d0_ref, d1_ref, g_ref, dinv_ref, dinv2_ref):
    deg = d0_ref[...] + d1_ref[...] + 1.0
    dinv = lax.rsqrt(deg)
    dinv_ref[...] = dinv
    dinv2_ref[...] = dinv * dinv
    g_ref[...] = jnp.dot(x_ref[...], w_ref[...],
                         preferred_element_type=_f32) * dinv


def _scale_body(p0_ref, p1_ref, d2_ref, o_ref):
    o_ref[...] = (p0_ref[...] + p1_ref[...]) * d2_ref[...]


def _mm2_body(q0_ref, q1_ref, dinv_ref, b_ref, w_ref, g_ref):
    dinv = dinv_ref[...]
    h = jnp.maximum((q0_ref[...] + q1_ref[...]) * dinv + b_ref[...], 0.0)
    g_ref[...] = jnp.dot(h, w_ref[...], preferred_element_type=_f32) * dinv


def _mm3_body(q0_ref, q1_ref, dinv_ref, b_ref, wfc_ref, bfc_ref,
              h2_ref, out_ref):
    h = jnp.maximum((q0_ref[...] + q1_ref[...]) * dinv_ref[...] + b_ref[...],
                    0.0)
    h2_ref[...] = h
    out_ref[...] = jnp.dot(h, wfc_ref[...],
                           preferred_element_type=_f32) + bfc_ref[...]


def _row_spec(FD):
    return pl.BlockSpec((BN, FD), lambda i: (i, 0))


def _row_spec_hi(FD):
    return pl.BlockSpec((BN, FD), lambda i: (i + GRID, 0))


def _full_spec(a, b):
    return pl.BlockSpec((a, b), lambda i: (0, 0))


def kernel(x, edge_index, W1, b1, W2, b2, Wfc, bfc):
    N, D = x.shape
    H1 = W1.shape[1]
    H2 = W2.shape[1]
    OUT = Wfc.shape[1]
    E = edge_index.shape[1]

    k = jnp.arange(EP - E, dtype=jnp.int32)
    ridx = jnp.concatenate([edge_index[0], k % N]).reshape(NW * NWIN, W)
    cidx = jnp.concatenate([edge_index[1], N + k % (N_SH - N)]
                           ).reshape(NW * NWIN, W)
    x_pad = jnp.zeros((N_SH, D), _f32).at[:N].set(x)
    z1 = jnp.zeros((N_SH, H1), _f32)
    z2 = jnp.zeros((N_SH, H2), _f32)
    b1r = b1.reshape(1, H1)
    b2r = b2.reshape(1, H2)
    bfcr = bfc.reshape(1, OUT)

    degf = _deg_kernel(cidx).reshape(2 * N_SH, 1)

    g1, dinv, dinv2 = pl.pallas_call(
        _mm1_body,
        grid=(GRID,),
        in_specs=[_row_spec(D), _full_spec(D, H1),
                  _row_spec(1), _row_spec_hi(1)],
        out_specs=[_row_spec(H1), _row_spec(1), _row_spec(1)],
        out_shape=[jax.ShapeDtypeStruct((N_SH, H1), _f32),
                   jax.ShapeDtypeStruct((N_SH, 1), _f32),
                   jax.ShapeDtypeStruct((N_SH, 1), _f32)],
    )(x_pad, W1, degf, degf)

    hop1 = _make_hop(H1)
    hop2 = _make_hop(H2)

    def scale(p, F):
        return pl.pallas_call(
            _scale_body,
            grid=(GRID,),
            in_specs=[_row_spec(F), _row_spec_hi(F), _row_spec(1)],
            out_specs=_row_spec(F),
            out_shape=jax.ShapeDtypeStruct((N_SH, F), _f32),
        )(p, p, dinv2)

    pA = hop1(g1, z1, ridx, cidx)
    g1b = scale(pA, H1)
    qA = hop1(g1b, z1, ridx, cidx)

    g2 = pl.pallas_call(
        _mm2_body,
        grid=(GRID,),
        in_specs=[_row_spec(H1), _row_spec_hi(H1), _row_spec(1),
                  _full_spec(1, H1), _full_spec(H1, H2)],
        out_specs=_row_spec(H2),
        out_shape=jax.ShapeDtypeStruct((N_SH, H2), _f32),
    )(qA, qA, dinv, b1r, W2)

    pB = hop2(g2, z2, ridx, cidx)
    g2b = scale(pB, H2)
    qB = hop2(g2b, z2, ridx, cidx)

    h2, out = pl.pallas_call(
        _mm3_body,
        grid=(GRID,),
        in_specs=[_row_spec(H2), _row_spec_hi(H2), _row_spec(1),
                  _full_spec(1, H2), _full_spec(H2, OUT), _full_spec(1, OUT)],
        out_specs=[_row_spec(H2), _row_spec(OUT)],
        out_shape=[jax.ShapeDtypeStruct((N_SH, H2), _f32),
                   jax.ShapeDtypeStruct((N_SH, OUT), _f32)],
    )(qB, qB, dinv, b2r, Wfc, bfcr)

    return h2[:N], out[:N]


# speedup vs baseline: 40.1784x; 1.1521x over previous
import functools

import jax
import jax.numpy as jnp
from jax import lax
from jax.experimental import pallas as pl
from jax.experimental.pallas import tpu as pltpu
from jax.experimental.pallas import tpu_sc as plsc

N_SH = 10240
W = 128
NC, NS = 2, 16
NW = NC * NS
NWIN = 80
EP = NW * NWIN * W
RPT = N_SH // NS
BN = 1024
GRID = N_SH // BN

_MESH = plsc.VectorSubcoreMesh(core_axis_name="c", subcore_axis_name="s",
                               num_cores=NC, num_subcores=NS)
_SC_PARAMS = pltpu.CompilerParams(use_tc_tiling_on_sc=False)
_f32 = jnp.float32


def _hop_body(g_hbm, z_hbm, ridx_hbm, cidx_hbm, out_hbm,
              ridx_v, cidx_v, rb0, rb1, rb2, rb3, agg_sh,
              sg0, sg1, sg2, sg3, ss0, ss1, ss2, ss3):
    c = lax.axis_index("c")
    s = lax.axis_index("s")
    wid = c * NS + s
    r0 = s * RPT
    rbs = (rb0, rb1, rb2, rb3)
    sgs = (sg0, sg1, sg2, sg3)
    sss = (ss0, ss1, ss2, ss3)

    @pl.when(c == 0)
    def _():
        pltpu.sync_copy(g_hbm.at[pl.ds(r0, RPT)], agg_sh.at[pl.ds(r0, RPT)])

    @pl.when(c != 0)
    def _():
        pltpu.sync_copy(z_hbm.at[pl.ds(r0, RPT)], agg_sh.at[pl.ds(r0, RPT)])

    pltpu.sync_copy(ridx_hbm.at[pl.ds(wid * NWIN, NWIN)], ridx_v)
    pltpu.sync_copy(cidx_hbm.at[pl.ds(wid * NWIN, NWIN)], cidx_v)

    pltpu.async_copy(g_hbm.at[ridx_v.at[0]], rb0, sg0)
    pltpu.async_copy(g_hbm.at[ridx_v.at[1]], rb1, sg1)
    plsc.subcore_barrier()

    def outer(i, carry):
        for b in range(4):
            w = 4 * i + b
            bp = (b + 2) % 4

            @pl.when(w >= 2)
            def _():
                pltpu.make_async_copy(
                    rbs[bp], agg_sh.at[cidx_v.at[w - 2]], sss[bp]).wait()

            @pl.when(w + 2 < NWIN)
            def _():
                pltpu.async_copy(g_hbm.at[ridx_v.at[w + 2]], rbs[bp], sgs[bp])

            pltpu.make_async_copy(g_hbm.at[ridx_v.at[w]], rbs[b],
                                  sgs[b]).wait()
            pltpu.async_copy(rbs[b], agg_sh.at[cidx_v.at[w]], sss[b],
                             add=True)
        return carry

    lax.fori_loop(0, NWIN // 4, outer, 0)
    for w in (NWIN - 2, NWIN - 1):
        pltpu.make_async_copy(rbs[w % 4], agg_sh.at[cidx_v.at[w]],
                              sss[w % 4]).wait()
    plsc.subcore_barrier()
    pltpu.sync_copy(agg_sh.at[pl.ds(r0, RPT)],
                    out_hbm.at[pl.ds(c * N_SH + r0, RPT)])


def _make_hop(F):
    return pl.kernel(
        _hop_body,
        out_type=jax.ShapeDtypeStruct((2 * N_SH, F), _f32),
        mesh=_MESH,
        scratch_types=[
            pltpu.VMEM((NWIN, W), jnp.int32),
            pltpu.VMEM((NWIN, W), jnp.int32),
            pltpu.VMEM((W, F), _f32),
            pltpu.VMEM((W, F), _f32),
            pltpu.VMEM((W, F), _f32),
            pltpu.VMEM((W, F), _f32),
            pltpu.VMEM_SHARED((N_SH, F), _f32),
        ] + [pltpu.SemaphoreType.DMA] * 8,
        compiler_params=_SC_PARAMS,
    )


def _deg_body(cidx_hbm, out_hbm, cidx_v, ones_v, zb_v, deg_sh):
    c = lax.axis_index("c")
    s = lax.axis_index("s")
    wid = c * NS + s
    r0 = s * RPT

    for j in range(W // 16):
        ones_v[pl.ds(j * 16, 16)] = jnp.ones((16,), _f32)

    def zf(i, carry):
        zb_v[pl.ds(i * 16, 16)] = jnp.zeros((16,), _f32)
        return carry

    lax.fori_loop(0, RPT // 16, zf, 0)
    pltpu.sync_copy(zb_v, deg_sh.at[pl.ds(r0, RPT)])
    pltpu.sync_copy(cidx_hbm.at[pl.ds(wid * NWIN, NWIN)], cidx_v)
    plsc.subcore_barrier()

    def wf(w, carry):
        pltpu.sync_copy(ones_v, deg_sh.at[cidx_v.at[w]], add=True)
        return carry

    lax.fori_loop(0, NWIN, wf, 0)
    plsc.subcore_barrier()
    pltpu.sync_copy(deg_sh.at[pl.ds(r0, RPT)],
                    out_hbm.at[pl.ds(c * N_SH + r0, RPT)])


_deg_kernel = pl.kernel(
    _deg_body,
    out_type=jax.ShapeDtypeStruct((2 * N_SH,), _f32),
    mesh=_MESH,
    scratch_types=[
        pltpu.VMEM((NWIN, W), jnp.int32),
        pltpu.VMEM((W,), _f32),
        pltpu.VMEM((RPT,), _f32),
        pltpu.VMEM_SHARED((N_SH,), _f32),
    ],
    compiler_params=_SC_PARAMS,
)


def _mm1_body(x_ref, w_ref, d0_ref, d1_ref, g_ref, dinv_ref, dinv2_ref):
    deg = d0_ref[...] + d1_ref[...] + 1.0
    dinv = lax.rsqrt(deg)
    dinv_ref[...] = dinv
    dinv2_ref[...] = dinv * dinv
    g_ref[...] = jnp.dot(x_ref[...], w_ref[...],
                         preferred_element_type=_f32) * dinv


def _scale_body(p0_ref, p1_ref, d2_ref, o_ref):
    o_ref[...] = (p0_ref[...] + p1_ref[...]) * d2_ref[...]


def _mm2_body(q0_ref, q1_ref, dinv_ref, b_ref, w_ref, g_ref):
    dinv = dinv_ref[...]
    h = jnp.maximum((q0_ref[...] + q1_ref[...]) * dinv + b_ref[...], 0.0)
    g_ref[...] = jnp.dot(h, w_ref[...], preferred_element_type=_f32) * dinv


def _mm3_body(q0_ref, q1_ref, dinv_ref, b_ref, wfc_ref, bfc_ref,
              h2_ref, out_ref):
    h = jnp.maximum((q0_ref[...] + q1_ref[...]) * dinv_ref[...] + b_ref[...],
                    0.0)
    h2_ref[...] = h
    out_ref[...] = jnp.dot(h, wfc_ref[...],
                           preferred_element_type=_f32) + bfc_ref[...]


def _row_spec(FD):
    return pl.BlockSpec((BN, FD), lambda i: (i, 0))


def _row_spec_hi(FD):
    return pl.BlockSpec((BN, FD), lambda i: (i + GRID, 0))


def _full_spec(a, b):
    return pl.BlockSpec((a, b), lambda i: (0, 0))


def kernel(x, edge_index, W1, b1, W2, b2, Wfc, bfc):
    N, D = x.shape
    H1 = W1.shape[1]
    H2 = W2.shape[1]
    OUT = Wfc.shape[1]
    E = edge_index.shape[1]

    k = jnp.arange(EP - E, dtype=jnp.int32)
    ridx = jnp.concatenate([edge_index[0], k % N]).reshape(NW * NWIN, W)
    cidx = jnp.concatenate([edge_index[1], N + k % (N_SH - N)]
                           ).reshape(NW * NWIN, W)
    z1 = jnp.zeros((N_SH, H1), _f32)
    z2 = jnp.zeros((N_SH, H2), _f32)
    b1r = b1.reshape(1, H1)
    b2r = b2.reshape(1, H2)
    bfcr = bfc.reshape(1, OUT)

    degf = _deg_kernel(cidx).reshape(2 * N_SH, 1)

    g1, dinv, dinv2 = pl.pallas_call(
        _mm1_body,
        grid=(GRID,),
        in_specs=[_row_spec(D), _full_spec(D, H1),
                  _row_spec(1), _row_spec_hi(1)],
        out_specs=[_row_spec(H1), _row_spec(1), _row_spec(1)],
        out_shape=[jax.ShapeDtypeStruct((N_SH, H1), _f32),
                   jax.ShapeDtypeStruct((N_SH, 1), _f32),
                   jax.ShapeDtypeStruct((N_SH, 1), _f32)],
    )(x, W1, degf, degf)

    hop1 = _make_hop(H1)
    hop2 = _make_hop(H2)

    def scale(p, F):
        return pl.pallas_call(
            _scale_body,
            grid=(GRID,),
            in_specs=[_row_spec(F), _row_spec_hi(F), _row_spec(1)],
            out_specs=_row_spec(F),
            out_shape=jax.ShapeDtypeStruct((N_SH, F), _f32),
        )(p, p, dinv2)

    pA = hop1(g1, z1, ridx, cidx)
    g1b = scale(pA, H1)
    qA = hop1(g1b, z1, ridx, cidx)

    g2 = pl.pallas_call(
        _mm2_body,
        grid=(GRID,),
        in_specs=[_row_spec(H1), _row_spec_hi(H1), _row_spec(1),
                  _full_spec(1, H1), _full_spec(H1, H2)],
        out_specs=_row_spec(H2),
        out_shape=jax.ShapeDtypeStruct((N_SH, H2), _f32),
    )(qA, qA, dinv, b1r, W2)

    pB = hop2(g2, z2, ridx, cidx)
    g2b = scale(pB, H2)
    qB = hop2(g2b, z2, ridx, cidx)

    h2, out = pl.pallas_call(
        _mm3_body,
        grid=(GRID,),
        in_specs=[_row_spec(H2), _row_spec_hi(H2), _row_spec(1),
                  _full_spec(1, H2), _full_spec(H2, OUT), _full_spec(1, OUT)],
        out_specs=[_row_spec(H2), _row_spec(OUT)],
        out_shape=[jax.ShapeDtypeStruct((N, H2), _f32),
                   jax.ShapeDtypeStruct((N, OUT), _f32)],
    )(qB, qB, dinv, b2r, Wfc, bfcr)

    return h2, out


# speedup vs baseline: 41.7843x; 1.0400x over previous
import functools

import numpy as np
import jax
import jax.numpy as jnp
from jax import lax
from jax.experimental import pallas as pl
from jax.experimental.pallas import tpu as pltpu
from jax.experimental.pallas import tpu_sc as plsc

N_SH = 10240
W = 128
NC, NS = 2, 16
NW = NC * NS
NWIN = 80
EP = NW * NWIN * W
RPT = N_SH // NS
BN = 1024
GRID = N_SH // BN

_MESH = plsc.VectorSubcoreMesh(core_axis_name="c", subcore_axis_name="s",
                               num_cores=NC, num_subcores=NS)
_SC_PARAMS = pltpu.CompilerParams(use_tc_tiling_on_sc=False)
_f32 = jnp.float32


def _hop_body(g_hbm, z_hbm, ridx_hbm, cidx_hbm, out_hbm,
              ridx_v, cidx_v, rb0, rb1, rb2, rb3, agg_sh,
              sg0, sg1, sg2, sg3, ss0, ss1, ss2, ss3):
    c = lax.axis_index("c")
    s = lax.axis_index("s")
    wid = c * NS + s
    r0 = s * RPT
    rbs = (rb0, rb1, rb2, rb3)
    sgs = (sg0, sg1, sg2, sg3)
    sss = (ss0, ss1, ss2, ss3)

    @pl.when(c == 0)
    def _():
        pltpu.sync_copy(g_hbm.at[pl.ds(r0, RPT)], agg_sh.at[pl.ds(r0, RPT)])

    @pl.when(c != 0)
    def _():
        pltpu.sync_copy(z_hbm.at[pl.ds(r0, RPT)], agg_sh.at[pl.ds(r0, RPT)])

    pltpu.sync_copy(ridx_hbm.at[pl.ds(wid * NWIN, NWIN)], ridx_v)
    pltpu.sync_copy(cidx_hbm.at[pl.ds(wid * NWIN, NWIN)], cidx_v)

    pltpu.async_copy(g_hbm.at[ridx_v.at[0]], rb0, sg0)
    pltpu.async_copy(g_hbm.at[ridx_v.at[1]], rb1, sg1)
    plsc.subcore_barrier()

    def outer(i, carry):
        for b in range(4):
            w = 4 * i + b
            bp = (b + 2) % 4

            @pl.when(w >= 2)
            def _():
                pltpu.make_async_copy(
                    rbs[bp], agg_sh.at[cidx_v.at[w - 2]], sss[bp]).wait()

            @pl.when(w + 2 < NWIN)
            def _():
                pltpu.async_copy(g_hbm.at[ridx_v.at[w + 2]], rbs[bp], sgs[bp])

            pltpu.make_async_copy(g_hbm.at[ridx_v.at[w]], rbs[b],
                                  sgs[b]).wait()
            pltpu.async_copy(rbs[b], agg_sh.at[cidx_v.at[w]], sss[b],
                             add=True)
        return carry

    lax.fori_loop(0, NWIN // 4, outer, 0)
    for w in (NWIN - 2, NWIN - 1):
        pltpu.make_async_copy(rbs[w % 4], agg_sh.at[cidx_v.at[w]],
                              sss[w % 4]).wait()
    plsc.subcore_barrier()
    pltpu.sync_copy(agg_sh.at[pl.ds(r0, RPT)],
                    out_hbm.at[pl.ds(c * N_SH + r0, RPT)])


def _make_hop(F):
    return pl.kernel(
        _hop_body,
        out_type=jax.ShapeDtypeStruct((2 * N_SH, F), _f32),
        mesh=_MESH,
        scratch_types=[
            pltpu.VMEM((NWIN, W), jnp.int32),
            pltpu.VMEM((NWIN, W), jnp.int32),
            pltpu.VMEM((W, F), _f32),
            pltpu.VMEM((W, F), _f32),
            pltpu.VMEM((W, F), _f32),
            pltpu.VMEM((W, F), _f32),
            pltpu.VMEM_SHARED((N_SH, F), _f32),
        ] + [pltpu.SemaphoreType.DMA] * 8,
        compiler_params=_SC_PARAMS,
    )


def _deg_body(cidx_hbm, out_hbm, cidx_v, ones_v, zb_v, deg_sh):
    c = lax.axis_index("c")
    s = lax.axis_index("s")
    wid = c * NS + s
    r0 = s * RPT

    for j in range(W // 16):
        ones_v[pl.ds(j * 16, 16)] = jnp.ones((16,), _f32)

    def zf(i, carry):
        zb_v[pl.ds(i * 16, 16)] = jnp.zeros((16,), _f32)
        return carry

    lax.fori_loop(0, RPT // 16, zf, 0)
    pltpu.sync_copy(zb_v, deg_sh.at[pl.ds(r0, RPT)])
    pltpu.sync_copy(cidx_hbm.at[pl.ds(wid * NWIN, NWIN)], cidx_v)
    plsc.subcore_barrier()

    def wf(w, carry):
        pltpu.sync_copy(ones_v, deg_sh.at[cidx_v.at[w]], add=True)
        return carry

    lax.fori_loop(0, NWIN, wf, 0)
    plsc.subcore_barrier()
    pltpu.sync_copy(deg_sh.at[pl.ds(r0, RPT)],
                    out_hbm.at[pl.ds(c * N_SH + r0, RPT)])


_deg_kernel = pl.kernel(
    _deg_body,
    out_type=jax.ShapeDtypeStruct((2 * N_SH,), _f32),
    mesh=_MESH,
    scratch_types=[
        pltpu.VMEM((NWIN, W), jnp.int32),
        pltpu.VMEM((W,), _f32),
        pltpu.VMEM((RPT,), _f32),
        pltpu.VMEM_SHARED((N_SH,), _f32),
    ],
    compiler_params=_SC_PARAMS,
)


def _deg_col(d0_ref, d1_ref):
    return jnp.reshape(d0_ref[...] + d1_ref[...] + 1.0, (BN, 1))


def _mm1_body(x_ref, w_ref, d0_ref, d1_ref, g_ref):
    dinv = lax.rsqrt(_deg_col(d0_ref, d1_ref))
    g_ref[...] = jnp.dot(x_ref[...], w_ref[...],
                         preferred_element_type=_f32) * dinv


def _scale_body(p0_ref, p1_ref, d0_ref, d1_ref, o_ref):
    o_ref[...] = (p0_ref[...] + p1_ref[...]) / _deg_col(d0_ref, d1_ref)


def _mm2_body(q0_ref, q1_ref, d0_ref, d1_ref, b_ref, w_ref, g_ref):
    dinv = lax.rsqrt(_deg_col(d0_ref, d1_ref))
    h = jnp.maximum((q0_ref[...] + q1_ref[...]) * dinv + b_ref[...], 0.0)
    g_ref[...] = jnp.dot(h, w_ref[...], preferred_element_type=_f32) * dinv


def _mm3_body(q0_ref, q1_ref, d0_ref, d1_ref, b_ref, wfc_ref, bfc_ref,
              h2_ref, out_ref):
    dinv = lax.rsqrt(_deg_col(d0_ref, d1_ref))
    h = jnp.maximum((q0_ref[...] + q1_ref[...]) * dinv + b_ref[...], 0.0)
    h2_ref[...] = h
    out_ref[...] = jnp.dot(h, wfc_ref[...],
                           preferred_element_type=_f32) + bfc_ref[...]


def _row_spec(FD):
    return pl.BlockSpec((BN, FD), lambda i: (i, 0))


def _row_spec_hi(FD):
    return pl.BlockSpec((BN, FD), lambda i: (i + GRID, 0))


def _full_spec(a, b):
    return pl.BlockSpec((a, b), lambda i: (0, 0))


def kernel(x, edge_index, W1, b1, W2, b2, Wfc, bfc):
    N, D = x.shape
    H1 = W1.shape[1]
    H2 = W2.shape[1]
    OUT = Wfc.shape[1]
    E = edge_index.shape[1]

    kpad = np.arange(EP - E, dtype=np.int32)
    ridx = jnp.concatenate(
        [edge_index[0], jnp.asarray(kpad % N)]).reshape(NW * NWIN, W)
    cidx = jnp.concatenate(
        [edge_index[1], jnp.asarray(N + kpad % (N_SH - N))]
    ).reshape(NW * NWIN, W)
    z1 = jnp.zeros((N_SH, H1), _f32)
    z2 = jnp.zeros((N_SH, H2), _f32)
    b1r = b1.reshape(1, H1)
    b2r = b2.reshape(1, H2)
    bfcr = bfc.reshape(1, OUT)

    degf = _deg_kernel(cidx)

    dspec = pl.BlockSpec((BN,), lambda i: (i,))
    dspec_hi = pl.BlockSpec((BN,), lambda i: (i + GRID,))

    g1 = pl.pallas_call(
        _mm1_body,
        grid=(GRID,),
        in_specs=[_row_spec(D), _full_spec(D, H1), dspec, dspec_hi],
        out_specs=_row_spec(H1),
        out_shape=jax.ShapeDtypeStruct((N_SH, H1), _f32),
    )(x, W1, degf, degf)

    hop1 = _make_hop(H1)
    hop2 = _make_hop(H2)

    def scale(p, F):
        return pl.pallas_call(
            _scale_body,
            grid=(GRID,),
            in_specs=[_row_spec(F), _row_spec_hi(F), dspec, dspec_hi],
            out_specs=_row_spec(F),
            out_shape=jax.ShapeDtypeStruct((N_SH, F), _f32),
        )(p, p, degf, degf)

    pA = hop1(g1, z1, ridx, cidx)
    g1b = scale(pA, H1)
    qA = hop1(g1b, z1, ridx, cidx)

    g2 = pl.pallas_call(
        _mm2_body,
        grid=(GRID,),
        in_specs=[_row_spec(H1), _row_spec_hi(H1), dspec, dspec_hi,
                  _full_spec(1, H1), _full_spec(H1, H2)],
        out_specs=_row_spec(H2),
        out_shape=jax.ShapeDtypeStruct((N_SH, H2), _f32),
    )(qA, qA, degf, degf, b1r, W2)

    pB = hop2(g2, z2, ridx, cidx)
    g2b = scale(pB, H2)
    qB = hop2(g2b, z2, ridx, cidx)

    h2, out = pl.pallas_call(
        _mm3_body,
        grid=(GRID,),
        in_specs=[_row_spec(H2), _row_spec_hi(H2), dspec, dspec_hi,
                  _full_spec(1, H2), _full_spec(H2, OUT), _full_spec(1, OUT)],
        out_specs=[_row_spec(H2), _row_spec(OUT)],
        out_shape=[jax.ShapeDtypeStruct((N, H2), _f32),
                   jax.ShapeDtypeStruct((N, OUT), _f32)],
    )(qB, qB, degf, degf, b2r, Wfc, bfcr)

    return h2, out


# speedup vs baseline: 45.6258x; 1.0919x over previous
import functools

import numpy as np
import jax
import jax.numpy as jnp
from jax import lax
from jax.experimental import pallas as pl
from jax.experimental.pallas import tpu as pltpu
from jax.experimental.pallas import tpu_sc as plsc

N_SH = 10240
W = 128
NC, NS = 2, 16
NW = NC * NS
NWIN = 80
EP = NW * NWIN * W
RPT = N_SH // NS
BN = 1024
GRID = N_SH // BN

_MESH = plsc.VectorSubcoreMesh(core_axis_name="c", subcore_axis_name="s",
                               num_cores=NC, num_subcores=NS)
_SC_PARAMS = pltpu.CompilerParams(use_tc_tiling_on_sc=False)
_f32 = jnp.float32


def _hop_body(g_hbm, z_hbm, ridx_hbm, cidx_hbm, out_hbm,
              ridx_v, cidx_v, rb0, rb1, rb2, rb3, agg_sh,
              sg0, sg1, sg2, sg3, ss0, ss1, ss2, ss3):
    c = lax.axis_index("c")
    s = lax.axis_index("s")
    wid = c * NS + s
    r0 = s * RPT
    rbs = (rb0, rb1, rb2, rb3)
    sgs = (sg0, sg1, sg2, sg3)
    sss = (ss0, ss1, ss2, ss3)

    @pl.when(c == 0)
    def _():
        pltpu.sync_copy(g_hbm.at[pl.ds(r0, RPT)], agg_sh.at[pl.ds(r0, RPT)])

    @pl.when(c != 0)
    def _():
        pltpu.sync_copy(z_hbm.at[pl.ds(r0, RPT)], agg_sh.at[pl.ds(r0, RPT)])

    pltpu.sync_copy(ridx_hbm.at[pl.ds(wid * NWIN, NWIN)], ridx_v)
    pltpu.sync_copy(cidx_hbm.at[pl.ds(wid * NWIN, NWIN)], cidx_v)

    pltpu.async_copy(g_hbm.at[ridx_v.at[0]], rb0, sg0)
    pltpu.async_copy(g_hbm.at[ridx_v.at[1]], rb1, sg1)
    plsc.subcore_barrier()

    def outer(i, carry):
        for b in range(4):
            w = 4 * i + b
            bp = (b + 2) % 4

            @pl.when(w >= 2)
            def _():
                pltpu.make_async_copy(
                    rbs[bp], agg_sh.at[cidx_v.at[w - 2]], sss[bp]).wait()

            @pl.when(w + 2 < NWIN)
            def _():
                pltpu.async_copy(g_hbm.at[ridx_v.at[w + 2]], rbs[bp], sgs[bp])

            pltpu.make_async_copy(g_hbm.at[ridx_v.at[w]], rbs[b],
                                  sgs[b]).wait()
            pltpu.async_copy(rbs[b], agg_sh.at[cidx_v.at[w]], sss[b],
                             add=True)
        return carry

    lax.fori_loop(0, NWIN // 4, outer, 0)
    for w in (NWIN - 2, NWIN - 1):
        pltpu.make_async_copy(rbs[w % 4], agg_sh.at[cidx_v.at[w]],
                              sss[w % 4]).wait()
    plsc.subcore_barrier()
    pltpu.sync_copy(agg_sh.at[pl.ds(r0, RPT)],
                    out_hbm.at[pl.ds(c * N_SH + r0, RPT)])


def _make_hop(F):
    return pl.kernel(
        _hop_body,
        out_type=jax.ShapeDtypeStruct((2 * N_SH, F), _f32),
        mesh=_MESH,
        scratch_types=[
            pltpu.VMEM((NWIN, W), jnp.int32),
            pltpu.VMEM((NWIN, W), jnp.int32),
            pltpu.VMEM((W, F), _f32),
            pltpu.VMEM((W, F), _f32),
            pltpu.VMEM((W, F), _f32),
            pltpu.VMEM((W, F), _f32),
            pltpu.VMEM_SHARED((N_SH, F), _f32),
        ] + [pltpu.SemaphoreType.DMA] * 8,
        compiler_params=_SC_PARAMS,
    )


def _deg_body(cidx_hbm, out_hbm, cidx_v, ones_v, zb_v, deg_sh):
    c = lax.axis_index("c")
    s = lax.axis_index("s")
    wid = c * NS + s
    r0 = s * RPT

    for j in range(W // 16):
        ones_v[pl.ds(j * 16, 16)] = jnp.ones((16,), _f32)

    def zf(i, carry):
        zb_v[pl.ds(i * 16, 16)] = jnp.zeros((16,), _f32)
        return carry

    lax.fori_loop(0, RPT // 16, zf, 0)
    pltpu.sync_copy(zb_v, deg_sh.at[pl.ds(r0, RPT)])
    pltpu.sync_copy(cidx_hbm.at[pl.ds(wid * NWIN, NWIN)], cidx_v)
    plsc.subcore_barrier()

    def wf(w, carry):
        pltpu.sync_copy(ones_v, deg_sh.at[cidx_v.at[w]], add=True)
        return carry

    lax.fori_loop(0, NWIN, wf, 0)
    plsc.subcore_barrier()
    pltpu.sync_copy(deg_sh.at[pl.ds(r0, RPT)],
                    out_hbm.at[pl.ds(c * N_SH + r0, RPT)])


_deg_kernel = pl.kernel(
    _deg_body,
    out_type=jax.ShapeDtypeStruct((2 * N_SH,), _f32),
    mesh=_MESH,
    scratch_types=[
        pltpu.VMEM((NWIN, W), jnp.int32),
        pltpu.VMEM((W,), _f32),
        pltpu.VMEM((RPT,), _f32),
        pltpu.VMEM_SHARED((N_SH,), _f32),
    ],
    compiler_params=_SC_PARAMS,
)


def _mm1_body(x_ref, w_ref, m_ref, g_ref):
    y = jnp.dot(x_ref[...], w_ref[...], preferred_element_type=_f32)
    g_ref[...] = y * m_ref[...]


def _scale_flat_body(p0_ref, p1_ref, m_ref, o_ref):
    m = m_ref[...]
    o_ref[...] = (p0_ref[...] + p1_ref[...]) * (m * m)


def _mm2_body(q0_ref, q1_ref, ma_ref, m32_ref, b_ref, w_ref, g_ref):
    ma = ma_ref[...]
    h = jnp.maximum((q0_ref[...] + q1_ref[...]) * ma + b_ref[...], 0.0)
    g = jnp.dot(h, w_ref[...], preferred_element_type=_f32)
    g_ref[...] = g * m32_ref[...]


def _mm3_body(q0_ref, q1_ref, m32_ref, b_ref, wfc_ref, bfc_ref,
              h2_ref, out_ref):
    h = jnp.maximum((q0_ref[...] + q1_ref[...]) * m32_ref[...] + b_ref[...],
                    0.0)
    h2_ref[...] = h
    out_ref[...] = jnp.dot(h, wfc_ref[...],
                           preferred_element_type=_f32) + bfc_ref[...]


def _p_spec(FD, off=0):
    return pl.BlockSpec((BN // 2, FD), lambda i: (i + off, 0))


def _full_spec(a, b):
    return pl.BlockSpec((a, b), lambda i: (0, 0))


def kernel(x, edge_index, W1, b1, W2, b2, Wfc, bfc):
    N, D = x.shape
    H1 = W1.shape[1]
    H2 = W2.shape[1]
    OUT = Wfc.shape[1]
    E = edge_index.shape[1]

    kpad = np.arange(EP - E, dtype=np.int32)
    ridx = jnp.concatenate(
        [edge_index[0], jnp.asarray(kpad % N)]).reshape(NW * NWIN, W)
    cidx = jnp.concatenate(
        [edge_index[1], jnp.asarray(N + kpad % (N_SH - N))]
    ).reshape(NW * NWIN, W)
    z1 = jnp.zeros((N_SH, H1), _f32)
    z2 = jnp.zeros((N_SH, H2), _f32)
    b1r = b1.reshape(1, H1)
    b2r = b2.reshape(1, H2)
    bfcr = bfc.reshape(1, OUT)

    degf = _deg_kernel(cidx)

    NH = N_SH // 2
    deg = degf[:N_SH] + degf[N_SH:] + 1.0
    dinv = lax.rsqrt(deg)
    ma2 = jnp.broadcast_to(dinv.reshape(NH, 2, 1), (NH, 2, H1)
                           ).reshape(NH, 2 * H1)
    maf = ma2.reshape(-1)
    m32 = jnp.broadcast_to(dinv.reshape(NH, 2, 1), (NH, 2, H2)
                           ).reshape(NH, 2 * H2)
    m32f = jnp.broadcast_to(dinv[:, None], (N_SH, H2)).reshape(-1)
    w1bd = jax.scipy.linalg.block_diag(W1, W1)
    w2bd = jax.scipy.linalg.block_diag(W2, W2)
    wfcbd = jax.scipy.linalg.block_diag(Wfc, Wfc)
    b1t = jnp.tile(b1, 2)[None]
    b2t = jnp.tile(b2, 2)[None]
    bfct = jnp.tile(bfc, 2)[None]
    xp = x.reshape(N // 2, 2 * D)

    g1p = pl.pallas_call(
        _mm1_body,
        grid=(GRID,),
        in_specs=[_p_spec(2 * D), _full_spec(2 * D, 2 * H1), _p_spec(2 * H1)],
        out_specs=_p_spec(2 * H1),
        out_shape=jax.ShapeDtypeStruct((NH, 2 * H1), _f32),
    )(xp, w1bd, ma2)

    hop1 = _make_hop(H1)
    hop2 = _make_hop(H2)

    def scale_flat(p, mf, nel):
        blk = nel // GRID
        return pl.pallas_call(
            _scale_flat_body,
            grid=(GRID,),
            in_specs=[pl.BlockSpec((blk,), lambda i: (i,)),
                      pl.BlockSpec((blk,), lambda i: (i + GRID,)),
                      pl.BlockSpec((blk,), lambda i: (i,))],
            out_specs=pl.BlockSpec((blk,), lambda i: (i,)),
            out_shape=jax.ShapeDtypeStruct((nel,), _f32),
        )(p, p, mf)

    NE1 = N_SH * H1
    NE2 = N_SH * H2
    pA = hop1(g1p.reshape(N_SH, H1), z1, ridx, cidx).reshape(2 * NE1)
    g1bf = scale_flat(pA, maf, NE1)
    qA = hop1(g1bf.reshape(N_SH, H1), z1, ridx, cidx).reshape(NH * 2, 128)

    g2p = pl.pallas_call(
        _mm2_body,
        grid=(GRID,),
        in_specs=[_p_spec(128), _p_spec(128, GRID), _p_spec(128),
                  _p_spec(2 * H2), _full_spec(1, 128),
                  _full_spec(2 * H1, 2 * H2)],
        out_specs=_p_spec(2 * H2),
        out_shape=jax.ShapeDtypeStruct((NH, 2 * H2), _f32),
    )(qA, qA, ma2, m32, b1t, w2bd)

    pB = hop2(g2p.reshape(N_SH, H2), z2, ridx, cidx).reshape(2 * NE2)
    g2bf = scale_flat(pB, m32f, NE2)
    qB = hop2(g2bf.reshape(N_SH, H2), z2, ridx, cidx).reshape(2 * NH, 2 * H2)

    h2p, outp = pl.pallas_call(
        _mm3_body,
        grid=(GRID,),
        in_specs=[_p_spec(2 * H2), _p_spec(2 * H2, GRID), _p_spec(2 * H2),
                  _full_spec(1, 2 * H2), _full_spec(2 * H2, 2 * OUT),
                  _full_spec(1, 2 * OUT)],
        out_specs=[_p_spec(2 * H2), _p_spec(2 * OUT)],
        out_shape=[jax.ShapeDtypeStruct((NH, 2 * H2), _f32),
                   jax.ShapeDtypeStruct((NH, 2 * OUT), _f32)],
    )(qB, qB, m32, b2t, wfcbd, bfct)

    return (h2p.reshape(N_SH, H2)[:N], outp.reshape(N_SH, OUT)[:N])


# speedup vs baseline: 48.8705x; 1.0711x over previous
import functools

import numpy as np
import jax
import jax.numpy as jnp
from jax import lax
from jax.experimental import pallas as pl
from jax.experimental.pallas import tpu as pltpu
from jax.experimental.pallas import tpu_sc as plsc

N_SH = 10240
W = 128
NC, NS = 2, 16
NW = NC * NS
NWIN = 80
EP = NW * NWIN * W
RPT = N_SH // NS
BN = 1024
GRID = N_SH // BN

_MESH = plsc.VectorSubcoreMesh(core_axis_name="c", subcore_axis_name="s",
                               num_cores=NC, num_subcores=NS)
_SC_PARAMS = pltpu.CompilerParams(use_tc_tiling_on_sc=False)
_f32 = jnp.float32


NB = 8
LOOK = NB // 2


def _hop_body(F, wide_out, g_hbm, z_hbm, ridx_hbm, cidx_hbm, out_hbm,
              ridx_v, cidx_v, *bufs):
    rbs = bufs[:NB]
    sgs = bufs[NB:2 * NB]
    sss = bufs[2 * NB:3 * NB]
    c = lax.axis_index("c")
    s = lax.axis_index("s")
    wid = c * NS + s
    r0 = s * RPT

    agg_sh = bufs[3 * NB]

    @pl.when(c == 0)
    def _():
        pltpu.sync_copy(g_hbm.at[pl.ds(r0, RPT)], agg_sh.at[pl.ds(r0, RPT)])

    @pl.when(c != 0)
    def _():
        pltpu.sync_copy(z_hbm.at[pl.ds(r0, RPT)], agg_sh.at[pl.ds(r0, RPT)])

    pltpu.sync_copy(ridx_hbm.at[pl.ds(wid * NWIN, NWIN)], ridx_v)
    pltpu.sync_copy(cidx_hbm.at[pl.ds(wid * NWIN, NWIN)], cidx_v)

    for w0 in range(LOOK):
        pltpu.async_copy(g_hbm.at[ridx_v.at[w0]], rbs[w0], sgs[w0])
    plsc.subcore_barrier()

    def outer(i, carry):
        for b in range(NB):
            w = NB * i + b
            bp = (b + LOOK) % NB

            @pl.when(w >= LOOK)
            def _():
                pltpu.make_async_copy(
                    rbs[bp], agg_sh.at[cidx_v.at[w - LOOK]], sss[bp]).wait()

            @pl.when(w + LOOK < NWIN)
            def _():
                pltpu.async_copy(g_hbm.at[ridx_v.at[w + LOOK]], rbs[bp],
                                 sgs[bp])

            pltpu.make_async_copy(g_hbm.at[ridx_v.at[w]], rbs[b],
                                  sgs[b]).wait()
            pltpu.async_copy(rbs[b], agg_sh.at[cidx_v.at[w]], sss[b],
                             add=True)
        return carry

    lax.fori_loop(0, NWIN // NB, outer, 0)
    for w in range(NWIN - LOOK, NWIN):
        pltpu.make_async_copy(rbs[w % NB], agg_sh.at[cidx_v.at[w]],
                              sss[w % NB]).wait()
    plsc.subcore_barrier()
    if wide_out:
        pltpu.sync_copy(agg_sh.at[pl.ds(r0, RPT)],
                        out_hbm.at[pl.ds(c * N_SH + r0, RPT), pl.ds(0, F)])
    else:
        pltpu.sync_copy(agg_sh.at[pl.ds(r0, RPT)],
                        out_hbm.at[pl.ds(c * N_SH + r0, RPT)])


def _make_hop(F, wide_out=False):
    ow = 128 if wide_out else F
    return pl.kernel(
        functools.partial(_hop_body, F, wide_out),
        out_type=jax.ShapeDtypeStruct((2 * N_SH, ow), _f32),
        mesh=_MESH,
        scratch_types=[
            pltpu.VMEM((NWIN, W), jnp.int32),
            pltpu.VMEM((NWIN, W), jnp.int32),
        ] + [pltpu.VMEM((W, F), _f32)] * NB
          + [pltpu.SemaphoreType.DMA] * (2 * NB)
          + [pltpu.VMEM_SHARED((N_SH, F), _f32)],
        compiler_params=_SC_PARAMS,
    )


def _deg_body(cidx_hbm, out_hbm, cidx_v, ones_v, zb_v, deg_sh):
    c = lax.axis_index("c")
    s = lax.axis_index("s")
    wid = c * NS + s
    r0 = s * RPT

    for j in range(W // 16):
        ones_v[pl.ds(j * 16, 16)] = jnp.ones((16,), _f32)

    def zf(i, carry):
        zb_v[pl.ds(i * 16, 16)] = jnp.zeros((16,), _f32)
        return carry

    lax.fori_loop(0, RPT // 16, zf, 0)
    pltpu.sync_copy(zb_v, deg_sh.at[pl.ds(r0, RPT)])
    pltpu.sync_copy(cidx_hbm.at[pl.ds(wid * NWIN, NWIN)], cidx_v)
    plsc.subcore_barrier()

    def wf(w, carry):
        pltpu.sync_copy(ones_v, deg_sh.at[cidx_v.at[w]], add=True)
        return carry

    lax.fori_loop(0, NWIN, wf, 0)
    plsc.subcore_barrier()
    pltpu.sync_copy(deg_sh.at[pl.ds(r0, RPT)],
                    out_hbm.at[pl.ds(c * N_SH + r0, RPT)])


_deg_kernel = pl.kernel(
    _deg_body,
    out_type=jax.ShapeDtypeStruct((2 * N_SH,), _f32),
    mesh=_MESH,
    scratch_types=[
        pltpu.VMEM((NWIN, W), jnp.int32),
        pltpu.VMEM((W,), _f32),
        pltpu.VMEM((RPT,), _f32),
        pltpu.VMEM_SHARED((N_SH,), _f32),
    ],
    compiler_params=_SC_PARAMS,
)


def _mm1_body(x_ref, w_ref, m_ref, g_ref):
    y = jnp.dot(x_ref[...], w_ref[...], preferred_element_type=_f32)
    g_ref[...] = y * m_ref[...]


def _scale_flat_body(p0_ref, p1_ref, m_ref, o_ref):
    m = m_ref[...]
    o_ref[...] = (p0_ref[...] + p1_ref[...]) * (m * m)


def _mm2_body(q0_ref, q1_ref, ma_ref, m32_ref, b_ref, w_ref, g_ref):
    ma = ma_ref[...]
    h = jnp.maximum((q0_ref[...] + q1_ref[...]) * ma + b_ref[...], 0.0)
    g = jnp.dot(h, w_ref[...], preferred_element_type=_f32)
    g_ref[...] = g * m32_ref[...]


def _mm3_body(q0_ref, q1_ref, d0_ref, d1_ref, b_ref, wfc_ref, bfc_ref,
              h2_ref, out_ref):
    F = b_ref.shape[1]
    dinv = lax.rsqrt(jnp.reshape(d0_ref[...] + d1_ref[...] + 1.0, (BN, 1)))
    q = q0_ref[:, :F] + q1_ref[:, :F]
    h = jnp.maximum(q * dinv + b_ref[...], 0.0)
    h2_ref[...] = h
    out_ref[...] = jnp.dot(h, wfc_ref[...],
                           preferred_element_type=_f32) + bfc_ref[...]


def _p_spec(FD, off=0):
    return pl.BlockSpec((BN // 2, FD), lambda i: (i + off, 0))


def _full_spec(a, b):
    return pl.BlockSpec((a, b), lambda i: (0, 0))


def kernel(x, edge_index, W1, b1, W2, b2, Wfc, bfc):
    N, D = x.shape
    H1 = W1.shape[1]
    H2 = W2.shape[1]
    OUT = Wfc.shape[1]
    E = edge_index.shape[1]

    kpad = np.arange(EP - E, dtype=np.int32)
    ridx = jnp.concatenate(
        [edge_index[0], jnp.asarray(kpad % N)]).reshape(NW * NWIN, W)
    cidx = jnp.concatenate(
        [edge_index[1], jnp.asarray(N + kpad % (N_SH - N))]
    ).reshape(NW * NWIN, W)
    z1 = jnp.zeros((N_SH, H1), _f32)
    z2 = jnp.zeros((N_SH, H2), _f32)
    b1r = b1.reshape(1, H1)
    b2r = b2.reshape(1, H2)
    bfcr = bfc.reshape(1, OUT)

    degf = _deg_kernel(cidx)

    NH = N_SH // 2
    deg = degf[:N_SH] + degf[N_SH:] + 1.0
    dinv = lax.rsqrt(deg)
    ma2 = jnp.broadcast_to(dinv.reshape(NH, 2, 1), (NH, 2, H1)
                           ).reshape(NH, 2 * H1)
    maf = ma2.reshape(-1)
    m32 = jnp.broadcast_to(dinv.reshape(NH, 2, 1), (NH, 2, H2)
                           ).reshape(NH, 2 * H2)
    m32f = jnp.broadcast_to(dinv[:, None], (N_SH, H2)).reshape(-1)
    w1bd = jax.scipy.linalg.block_diag(W1, W1)
    w2bd = jax.scipy.linalg.block_diag(W2, W2)
    wfcbd = jax.scipy.linalg.block_diag(Wfc, Wfc)
    b1t = jnp.tile(b1, 2)[None]
    b2t = jnp.tile(b2, 2)[None]
    bfct = jnp.tile(bfc, 2)[None]
    xp = x.reshape(N // 2, 2 * D)

    g1p = pl.pallas_call(
        _mm1_body,
        grid=(GRID,),
        in_specs=[_p_spec(2 * D), _full_spec(2 * D, 2 * H1), _p_spec(2 * H1)],
        out_specs=_p_spec(2 * H1),
        out_shape=jax.ShapeDtypeStruct((NH, 2 * H1), _f32),
    )(xp, w1bd, ma2)

    hop1 = _make_hop(H1)
    hop2 = _make_hop(H2)
    hop2w = _make_hop(H2, wide_out=True)

    def scale_flat(p, mf, nel):
        blk = nel // GRID
        return pl.pallas_call(
            _scale_flat_body,
            grid=(GRID,),
            in_specs=[pl.BlockSpec((blk,), lambda i: (i,)),
                      pl.BlockSpec((blk,), lambda i: (i + GRID,)),
                      pl.BlockSpec((blk,), lambda i: (i,))],
            out_specs=pl.BlockSpec((blk,), lambda i: (i,)),
            out_shape=jax.ShapeDtypeStruct((nel,), _f32),
        )(p, p, mf)

    NE1 = N_SH * H1
    NE2 = N_SH * H2
    pA = hop1(g1p.reshape(N_SH, H1), z1, ridx, cidx).reshape(2 * NE1)
    g1bf = scale_flat(pA, maf, NE1)
    qA = hop1(g1bf.reshape(N_SH, H1), z1, ridx, cidx).reshape(NH * 2, 128)

    g2p = pl.pallas_call(
        _mm2_body,
        grid=(GRID,),
        in_specs=[_p_spec(128), _p_spec(128, GRID), _p_spec(128),
                  _p_spec(2 * H2), _full_spec(1, 128),
                  _full_spec(2 * H1, 2 * H2)],
        out_specs=_p_spec(2 * H2),
        out_shape=jax.ShapeDtypeStruct((NH, 2 * H2), _f32),
    )(qA, qA, ma2, m32, b1t, w2bd)

    pB = hop2(g2p.reshape(N_SH, H2), z2, ridx, cidx).reshape(2 * NE2)
    g2bf = scale_flat(pB, m32f, NE2)
    qB = hop2w(g2bf.reshape(N_SH, H2), z2, ridx, cidx)

    dspec = pl.BlockSpec((BN,), lambda i: (i,))
    dspec_hi = pl.BlockSpec((BN,), lambda i: (i + GRID,))
    h2, out = pl.pallas_call(
        _mm3_body,
        grid=(GRID,),
        in_specs=[pl.BlockSpec((BN, 128), lambda i: (i, 0)),
                  pl.BlockSpec((BN, 128), lambda i: (i + GRID, 0)),
                  dspec, dspec_hi, _full_spec(1, H2), _full_spec(H2, OUT),
                  _full_spec(1, OUT)],
        out_specs=[pl.BlockSpec((BN, H2), lambda i: (i, 0)),
                   pl.BlockSpec((BN, OUT), lambda i: (i, 0))],
        out_shape=[jax.ShapeDtypeStruct((N, H2), _f32),
                   jax.ShapeDtypeStruct((N, OUT), _f32)],
    )(qB, qB, degf, degf, b2.reshape(1, H2), Wfc, bfc.reshape(1, OUT))

    return h2, out
